# Initial kernel scaffold; baseline (speedup 1.0000x reference)
import jax
import jax.numpy as jnp
from jax.experimental import pallas as pl


def kernel(hidden_states, attractors):
    raise NotImplementedError("write your pallas kernel here")


# speedup vs baseline: 1.0421x; 1.0421x over previous
import jax
import jax.numpy as jnp
from jax import lax
from jax.experimental import pallas as pl
from jax.experimental.pallas import tpu as pltpu

ALPHA = 0.3
_BS = 512


def _body(hid_ref, attr_ref, out_ref):
    out_ref[...] = hid_ref[...]

    nb = pl.num_programs(1)
    j = pl.program_id(1)

    @pl.when(j == nb - 1)
    def _update():
        h = hid_ref[0, _BS - 1, :].reshape(1, -1)
        norm = jnp.sqrt(jnp.sum(h * h))
        safe = jnp.maximum(norm, 1e-12)
        h_n = h / safe
        sims = lax.dot_general(
            h_n, attr_ref[...], (((1,), (1,)), ((), ())),
            preferred_element_type=jnp.float32)
        k = sims.shape[1]
        iota = lax.broadcasted_iota(jnp.int32, (1, k), 1)
        m = jnp.max(sims)
        idx = jnp.min(jnp.where(sims == m, iota, k))
        one_hot = (iota == idx).astype(jnp.float32)
        nearest = lax.dot_general(
            one_hot, attr_ref[...], (((1,), (0,)), ((), ())),
            preferred_element_type=jnp.float32)
        out_ref[0, _BS - 1, :] = (h + ALPHA * (nearest - h_n) * norm)[0]


def kernel(hidden_states, attractors):
    b, s, d = hidden_states.shape
    grid = (b, s // _BS)
    return pl.pallas_call(
        _body,
        grid=grid,
        in_specs=[
            pl.BlockSpec((1, _BS, d), lambda i, j: (i, j, 0)),
            pl.BlockSpec((attractors.shape[0], d), lambda i, j: (0, 0)),
        ],
        out_specs=pl.BlockSpec((1, _BS, d), lambda i, j: (i, j, 0)),
        out_shape=jax.ShapeDtypeStruct((b, s, d), hidden_states.dtype),
        compiler_params=pltpu.CompilerParams(
            dimension_semantics=("arbitrary", "arbitrary"),
        ),
    )(hidden_states, attractors)


# speedup vs baseline: 1.0799x; 1.0363x over previous
import jax
import jax.numpy as jnp
from jax import lax
from jax.experimental import pallas as pl
from jax.experimental.pallas import tpu as pltpu

ALPHA = 0.3
_BS = 1024


def _body(hid_ref, attr_ref, out_ref):
    out_ref[...] = hid_ref[...]

    nb = pl.num_programs(1)
    j = pl.program_id(1)

    @pl.when(j == nb - 1)
    def _update():
        h = hid_ref[0, _BS - 1, :].reshape(1, -1)
        norm = jnp.sqrt(jnp.sum(h * h))
        safe = jnp.maximum(norm, 1e-12)
        h_n = h / safe
        sims = lax.dot_general(
            h_n, attr_ref[...], (((1,), (1,)), ((), ())),
            preferred_element_type=jnp.float32)
        k = sims.shape[1]
        iota = lax.broadcasted_iota(jnp.int32, (1, k), 1)
        m = jnp.max(sims)
        idx = jnp.min(jnp.where(sims == m, iota, k))
        one_hot = (iota == idx).astype(jnp.float32)
        nearest = lax.dot_general(
            one_hot, attr_ref[...], (((1,), (0,)), ((), ())),
            preferred_element_type=jnp.float32)
        out_ref[0, _BS - 1, :] = (h + ALPHA * (nearest - h_n) * norm)[0]


def kernel(hidden_states, attractors):
    b, s, d = hidden_states.shape
    grid = (b, s // _BS)
    return pl.pallas_call(
        _body,
        grid=grid,
        in_specs=[
            pl.BlockSpec((1, _BS, d), lambda i, j: (i, j, 0)),
            pl.BlockSpec((attractors.shape[0], d), lambda i, j: (0, 0)),
        ],
        out_specs=pl.BlockSpec((1, _BS, d), lambda i, j: (i, j, 0)),
        out_shape=jax.ShapeDtypeStruct((b, s, d), hidden_states.dtype),
        compiler_params=pltpu.CompilerParams(
            dimension_semantics=("arbitrary", "arbitrary"),
        ),
    )(hidden_states, attractors)
